# Initial kernel scaffold; baseline (speedup 1.0000x reference)
import jax
import jax.numpy as jnp
from jax.experimental import pallas as pl


def kernel(I, T, L, S, B, A):
    raise NotImplementedError("write your pallas kernel here")


# speedup vs baseline: 1.3549x; 1.3549x over previous
import jax
import jax.numpy as jnp
from jax.experimental import pallas as pl

_C = 64
_K = 16
_DEPTH = 4
_NODES = _K - 1
_BN = 512


def _halut_block_kernel(i_ref, a_ref, t_ref, l_ref, o_ref):
    ia = jnp.dot(i_ref[...], a_ref[...], preferred_element_type=jnp.float32)
    ia = ia.astype(jnp.bfloat16).astype(jnp.float32)

    d = []
    for i in range(_NODES):
        lvl = (i + 1).bit_length() - 1
        h = ia[:, lvl * _C:(lvl + 1) * _C]
        di = jnp.tanh(h - t_ref[i, :][None, :])
        d.append(di.astype(jnp.bfloat16).astype(jnp.float32))

    p1 = [-d[0], d[0]]
    p2, p3, bs = [], [], []
    for j in range(4):
        b0, b1 = (j >> 1) & 1, j & 1
        n1 = 1 + b0
        p2.append(p1[b0] + (d[n1] if b1 else -d[n1]))
    for j in range(8):
        b0, b1, b2 = (j >> 2) & 1, (j >> 1) & 1, j & 1
        n2 = 3 + 2 * b0 + b1
        p3.append(p2[j >> 1] + (d[n2] if b2 else -d[n2]))
    for k in range(_K):
        b0, b1, b2, b3 = (k >> 3) & 1, (k >> 2) & 1, (k >> 1) & 1, k & 1
        n3 = 7 + 4 * b0 + 2 * b1 + b2
        bs.append(p3[k >> 1] + (d[n3] if b3 else -d[n3]))

    best_v = bs[0]
    best_k = jnp.zeros(bs[0].shape, dtype=jnp.int32)
    for k in range(1, _K):
        upd = bs[k] > best_v
        best_v = jnp.where(upd, bs[k], best_v)
        best_k = jnp.where(upd, jnp.int32(k), best_k)

    codes_t = jnp.concatenate([best_k] * _K, axis=1)
    kk = jax.lax.broadcasted_iota(jnp.int32, codes_t.shape, 1) // _C
    e = (codes_t == kk).astype(l_ref.dtype)
    o_ref[...] = jnp.dot(e, l_ref[...], preferred_element_type=jnp.float32)


def kernel(I, T, L, S, B, A):
    del S, B
    n, dim = I.shape
    m = L.shape[0]
    a_perm = (A.reshape(dim, _C, _DEPTH).transpose(0, 2, 1)
              .reshape(dim, _C * _DEPTH).astype(jnp.bfloat16))
    i_bf = I.astype(jnp.bfloat16)
    t_perm = jnp.pad(T.reshape(_C, _NODES).T, ((0, 1), (0, 0)))
    l4 = L.transpose(2, 1, 0).reshape(_K * _C, m).astype(jnp.bfloat16)

    return pl.pallas_call(
        _halut_block_kernel,
        grid=(n // _BN,),
        in_specs=[
            pl.BlockSpec((_BN, dim), lambda i: (i, 0)),
            pl.BlockSpec((dim, _C * _DEPTH), lambda i: (0, 0)),
            pl.BlockSpec((_K, _C), lambda i: (0, 0)),
            pl.BlockSpec((_K * _C, m), lambda i: (0, 0)),
        ],
        out_specs=pl.BlockSpec((_BN, m), lambda i: (i, 0)),
        out_shape=jax.ShapeDtypeStruct((n, m), jnp.float32),
    )(i_bf, a_perm, t_perm, l4)


# speedup vs baseline: 1.4126x; 1.0426x over previous
import numpy as np

import jax
import jax.numpy as jnp
from jax.experimental import pallas as pl

_C = 64
_K = 16
_DEPTH = 4
_NODES = _K - 1
_BN = 512
_HALF = _BN // 2


def _halut_block_kernel(i_ref, a_ref, t_ref, tile_ref, kk_ref, l_ref, o_ref):
    f32 = jnp.float32
    ia0 = jnp.dot(i_ref[:_HALF], a_ref[...], preferred_element_type=f32)
    ia1 = jnp.dot(i_ref[_HALF:], a_ref[...], preferred_element_type=f32)

    hs = []
    for lvl in range(_DEPTH):
        sl = slice(lvl * _C, (lvl + 1) * _C)
        h = jnp.concatenate([ia0[:, sl], ia1[:, sl]], axis=1)
        hs.append(h.astype(jnp.bfloat16).astype(f32))

    d = []
    for i in range(_NODES):
        lvl = (i + 1).bit_length() - 1
        di = jnp.tanh(hs[lvl] - t_ref[i, :][None, :])
        d.append(di.astype(jnp.bfloat16).astype(f32))

    p1 = [-d[0], d[0]]
    p2, p3, bs = [], [], []
    for j in range(4):
        b0, b1 = (j >> 1) & 1, j & 1
        n1 = 1 + b0
        p2.append(p1[b0] + (d[n1] if b1 else -d[n1]))
    for j in range(8):
        b0, b1, b2 = (j >> 2) & 1, (j >> 1) & 1, j & 1
        n2 = 3 + 2 * b0 + b1
        p3.append(p2[j >> 1] + (d[n2] if b2 else -d[n2]))
    for k in range(_K):
        b0, b1, b2 = (k >> 3) & 1, (k >> 2) & 1, (k >> 1) & 1
        n3 = 7 + 4 * b0 + 2 * b1 + b2
        bs.append(p3[k >> 1] + (d[n3] if (k & 1) else -d[n3]))

    best_v = bs[0]
    best_k = jnp.zeros(bs[0].shape, dtype=f32)
    for k in range(1, _K):
        upd = bs[k] > best_v
        best_v = jnp.where(upd, bs[k], best_v)
        best_k = jnp.where(upd, f32(k), best_k)

    codes = best_k.astype(jnp.bfloat16)

    for half, codes_h in ((0, codes[:, :_C]), (1, codes[:, _C:])):
        codes_t = jnp.dot(codes_h, tile_ref[...], preferred_element_type=f32)
        e = (codes_t == kk_ref[0:1, :]).astype(jnp.bfloat16)
        out = jnp.dot(e, l_ref[...], preferred_element_type=f32)
        o_ref[half * _HALF:(half + 1) * _HALF] = out


def kernel(I, T, L, S, B, A):
    del S, B
    n, dim = I.shape
    m = L.shape[0]
    a_perm = (A.reshape(dim, _C, _DEPTH).transpose(0, 2, 1)
              .reshape(dim, _C * _DEPTH).astype(jnp.bfloat16))
    i_bf = I.astype(jnp.bfloat16)
    t15 = T.reshape(_C, _NODES).T
    t_perm = jnp.pad(jnp.concatenate([t15, t15], axis=1), ((0, 1), (0, 0)))
    l4 = L.transpose(2, 1, 0).reshape(_K * _C, m).astype(jnp.bfloat16)
    tile = jnp.asarray(np.tile(np.eye(_C, dtype=np.float32), (1, _K)),
                       dtype=jnp.bfloat16)
    kk = jnp.asarray(np.broadcast_to(
        (np.arange(_K * _C) // _C).astype(np.float32), (8, _K * _C)))

    return pl.pallas_call(
        _halut_block_kernel,
        grid=(n // _BN,),
        in_specs=[
            pl.BlockSpec((_BN, dim), lambda i: (i, 0)),
            pl.BlockSpec((dim, _C * _DEPTH), lambda i: (0, 0)),
            pl.BlockSpec((_K, 2 * _C), lambda i: (0, 0)),
            pl.BlockSpec((_C, _K * _C), lambda i: (0, 0)),
            pl.BlockSpec((8, _K * _C), lambda i: (0, 0)),
            pl.BlockSpec((_K * _C, m), lambda i: (0, 0)),
        ],
        out_specs=pl.BlockSpec((_BN, m), lambda i: (i, 0)),
        out_shape=jax.ShapeDtypeStruct((n, m), jnp.float32),
    )(i_bf, a_perm, t_perm, tile, kk, l4)


# speedup vs baseline: 1.6690x; 1.1815x over previous
import numpy as np

import jax
import jax.numpy as jnp
from jax.experimental import pallas as pl

_C = 64
_K = 16
_DEPTH = 4
_NODES = _K - 1
_BN = 512
_HALF = _BN // 2


def _halut_block_kernel(i_ref, a_ref, t_ref, tile_ref, kk_ref, l_ref, o_ref):
    f32 = jnp.float32
    ib = i_ref[...].astype(jnp.bfloat16)
    ia0 = jnp.dot(ib[:_HALF], a_ref[...], preferred_element_type=f32)
    ia1 = jnp.dot(ib[_HALF:], a_ref[...], preferred_element_type=f32)

    hs = []
    for lvl in range(_DEPTH):
        sl = slice(lvl * _C, (lvl + 1) * _C)
        h = jnp.concatenate([ia0[:, sl], ia1[:, sl]], axis=1)
        hs.append(h.astype(jnp.bfloat16).astype(f32))

    d = []
    for i in range(_NODES):
        lvl = (i + 1).bit_length() - 1
        di = jnp.tanh(hs[lvl] - t_ref[i, :][None, :])
        d.append(di.astype(jnp.bfloat16).astype(f32))

    p1 = [-d[0], d[0]]
    p2, p3, bs = [], [], []
    for j in range(4):
        b0, b1 = (j >> 1) & 1, j & 1
        n1 = 1 + b0
        p2.append(p1[b0] + (d[n1] if b1 else -d[n1]))
    for j in range(8):
        b0, b1, b2 = (j >> 2) & 1, (j >> 1) & 1, j & 1
        n2 = 3 + 2 * b0 + b1
        p3.append(p2[j >> 1] + (d[n2] if b2 else -d[n2]))
    for k in range(_K):
        b0, b1, b2 = (k >> 3) & 1, (k >> 2) & 1, (k >> 1) & 1
        n3 = 7 + 4 * b0 + 2 * b1 + b2
        bs.append(p3[k >> 1] + (d[n3] if (k & 1) else -d[n3]))

    best_v = bs[0]
    best_k = jnp.zeros(bs[0].shape, dtype=f32)
    for k in range(1, _K):
        upd = bs[k] > best_v
        best_v = jnp.where(upd, bs[k], best_v)
        best_k = jnp.where(upd, f32(k), best_k)

    codes = best_k.astype(jnp.bfloat16)

    for half, codes_h in ((0, codes[:, :_C]), (1, codes[:, _C:])):
        codes_t = jnp.dot(codes_h, tile_ref[...], preferred_element_type=f32)
        e = (codes_t == kk_ref[0:1, :]).astype(jnp.bfloat16)
        out = jax.lax.dot_general(
            e, l_ref[...], (((1,), (1,)), ((), ())),
            preferred_element_type=f32)
        o_ref[half * _HALF:(half + 1) * _HALF] = out


def kernel(I, T, L, S, B, A):
    del S, B
    n, dim = I.shape
    m = L.shape[0]
    a_perm = (A.reshape(dim, _C, _DEPTH).transpose(0, 2, 1)
              .reshape(dim, _C * _DEPTH).astype(jnp.bfloat16))
    t15 = T.reshape(_C, _NODES).T
    t_perm = jnp.pad(jnp.concatenate([t15, t15], axis=1), ((0, 1), (0, 0)))
    l4 = L.reshape(m, _C * _K).astype(jnp.bfloat16)
    tile = jnp.asarray(np.repeat(np.eye(_C, dtype=np.float32), _K, axis=1),
                       dtype=jnp.bfloat16)
    kk = jnp.asarray(np.broadcast_to(
        (np.arange(_K * _C) % _K).astype(np.float32), (8, _K * _C)))

    return pl.pallas_call(
        _halut_block_kernel,
        grid=(n // _BN,),
        in_specs=[
            pl.BlockSpec((_BN, dim), lambda i: (i, 0)),
            pl.BlockSpec((dim, _C * _DEPTH), lambda i: (0, 0)),
            pl.BlockSpec((_K, 2 * _C), lambda i: (0, 0)),
            pl.BlockSpec((_C, _K * _C), lambda i: (0, 0)),
            pl.BlockSpec((8, _K * _C), lambda i: (0, 0)),
            pl.BlockSpec((m, _K * _C), lambda i: (0, 0)),
        ],
        out_specs=pl.BlockSpec((_BN, m), lambda i: (i, 0)),
        out_shape=jax.ShapeDtypeStruct((n, m), jnp.float32),
    )(I, a_perm, t_perm, tile, kk, l4)


# speedup vs baseline: 1.7140x; 1.0270x over previous
import numpy as np

import jax
import jax.numpy as jnp
from jax.experimental import pallas as pl

_C = 64
_K = 16
_DEPTH = 4
_NODES = _K - 1
_BN = 1024
_HALF = _BN // 2


def _halut_block_kernel(i_ref, a_ref, t_ref, tile_ref, kk_ref, l_ref, o_ref):
    f32 = jnp.float32
    ib = i_ref[...].astype(jnp.bfloat16)
    ia0 = jnp.dot(ib[:_HALF], a_ref[...], preferred_element_type=f32)
    ia1 = jnp.dot(ib[_HALF:], a_ref[...], preferred_element_type=f32)

    hs = []
    for lvl in range(_DEPTH):
        sl = slice(lvl * _C, (lvl + 1) * _C)
        h = jnp.concatenate([ia0[:, sl], ia1[:, sl]], axis=1)
        hs.append(h.astype(jnp.bfloat16).astype(f32))

    d = []
    for i in range(_NODES):
        lvl = (i + 1).bit_length() - 1
        di = jnp.tanh(hs[lvl] - t_ref[i, :][None, :])
        d.append(di.astype(jnp.bfloat16).astype(f32))

    p1 = [-d[0], d[0]]
    p2, p3, bs = [], [], []
    for j in range(4):
        b0, b1 = (j >> 1) & 1, j & 1
        n1 = 1 + b0
        p2.append(p1[b0] + (d[n1] if b1 else -d[n1]))
    for j in range(8):
        b0, b1, b2 = (j >> 2) & 1, (j >> 1) & 1, j & 1
        n2 = 3 + 2 * b0 + b1
        p3.append(p2[j >> 1] + (d[n2] if b2 else -d[n2]))
    for k in range(_K):
        b0, b1, b2 = (k >> 3) & 1, (k >> 2) & 1, (k >> 1) & 1
        n3 = 7 + 4 * b0 + 2 * b1 + b2
        bs.append(p3[k >> 1] + (d[n3] if (k & 1) else -d[n3]))

    best_v = bs[0]
    best_k = jnp.zeros(bs[0].shape, dtype=f32)
    for k in range(1, _K):
        upd = bs[k] > best_v
        best_v = jnp.where(upd, bs[k], best_v)
        best_k = jnp.where(upd, f32(k), best_k)

    codes = best_k.astype(jnp.bfloat16)

    for half, codes_h in ((0, codes[:, :_C]), (1, codes[:, _C:])):
        codes_t = jnp.dot(codes_h, tile_ref[...], preferred_element_type=f32)
        e = (codes_t == kk_ref[0:1, :]).astype(jnp.bfloat16)
        out = jax.lax.dot_general(
            e, l_ref[...], (((1,), (1,)), ((), ())),
            preferred_element_type=f32)
        o_ref[half * _HALF:(half + 1) * _HALF] = out


def kernel(I, T, L, S, B, A):
    del S, B
    n, dim = I.shape
    m = L.shape[0]
    a_perm = (A.reshape(dim, _C, _DEPTH).transpose(0, 2, 1)
              .reshape(dim, _C * _DEPTH).astype(jnp.bfloat16))
    t15 = T.reshape(_C, _NODES).T
    t_perm = jnp.pad(jnp.concatenate([t15, t15], axis=1), ((0, 1), (0, 0)))
    l4 = L.reshape(m, _C * _K).astype(jnp.bfloat16)
    tile = jnp.asarray(np.repeat(np.eye(_C, dtype=np.float32), _K, axis=1),
                       dtype=jnp.bfloat16)
    kk = jnp.asarray(np.broadcast_to(
        (np.arange(_K * _C) % _K).astype(np.float32), (8, _K * _C)))

    return pl.pallas_call(
        _halut_block_kernel,
        grid=(n // _BN,),
        in_specs=[
            pl.BlockSpec((_BN, dim), lambda i: (i, 0)),
            pl.BlockSpec((dim, _C * _DEPTH), lambda i: (0, 0)),
            pl.BlockSpec((_K, 2 * _C), lambda i: (0, 0)),
            pl.BlockSpec((_C, _K * _C), lambda i: (0, 0)),
            pl.BlockSpec((8, _K * _C), lambda i: (0, 0)),
            pl.BlockSpec((m, _K * _C), lambda i: (0, 0)),
        ],
        out_specs=pl.BlockSpec((_BN, m), lambda i: (i, 0)),
        out_shape=jax.ShapeDtypeStruct((n, m), jnp.float32),
    )(I, a_perm, t_perm, tile, kk, l4)
